# Initial kernel scaffold; baseline (speedup 1.0000x reference)
import jax
import jax.numpy as jnp
from jax.experimental import pallas as pl


def kernel(topology_features, geometry_features, extrusion_features, top_W1, top_b1, top_W2, top_b2, geo_W1, geo_b1, geo_W2, geo_b2, ext_W1, ext_b1, ext_W2, ext_b2, top_codebook, geo_codebook, ext_codebook):
    raise NotImplementedError("write your pallas kernel here")


# speedup vs baseline: 2.0883x; 2.0883x over previous
import functools

import jax
import jax.numpy as jnp
from jax import lax
from jax.experimental import pallas as pl
from jax.experimental.pallas import tpu as pltpu
from jax.experimental.pallas import tpu_sc as plsc

B = 16384
D = 256
TB = 512


def _vq_body(n, K, x_ref, w1_ref, b1_ref, w2_ref, b2_ref, cb_ref,
             idx_ref, loss_ref):
    i = pl.program_id(0)
    x = x_ref[...]
    h = jnp.maximum(
        jnp.dot(x, w1_ref[...], preferred_element_type=jnp.float32)
        + b1_ref[...], 0.0)
    z = (jnp.dot(h, w2_ref[...], preferred_element_type=jnp.float32)
         + b2_ref[...])
    cb = cb_ref[...]
    cbn = jnp.sum(cb * cb, axis=1)
    acc = jnp.zeros((), jnp.float32)
    cols = []
    for j in range(n):
        zj = z[:, j * D:(j + 1) * D]
        zn = jnp.sum(zj * zj, axis=1, keepdims=True)
        mm = lax.dot_general(zj, cb, (((1,), (1,)), ((), ())),
                             preferred_element_type=jnp.float32)
        d = zn + cbn[None, :] - 2.0 * mm
        m = jnp.min(d, axis=1, keepdims=True)
        iota = lax.broadcasted_iota(jnp.int32, d.shape, 1)
        aj = jnp.min(jnp.where(d == m, iota, K), axis=1, keepdims=True)
        cols.append(aj)
        acc = acc + jnp.sum(m)
    idx_ref[...] = jnp.concatenate(cols, axis=1)

    @pl.when(i == 0)
    def _init():
        loss_ref[0, 0] = 0.0

    loss_ref[0, 0] += acc


def _stream_tc(x, W1, b1, W2, b2, cb, n, K):
    grid = (B // TB,)
    idx, losssum = pl.pallas_call(
        functools.partial(_vq_body, n, K),
        grid=grid,
        in_specs=[
            pl.BlockSpec((TB, D), lambda i: (i, 0)),
            pl.BlockSpec((D, D), lambda i: (0, 0)),
            pl.BlockSpec((1, D), lambda i: (0, 0)),
            pl.BlockSpec((D, n * D), lambda i: (0, 0)),
            pl.BlockSpec((1, n * D), lambda i: (0, 0)),
            pl.BlockSpec((K, D), lambda i: (0, 0)),
        ],
        out_specs=[
            pl.BlockSpec((TB, n), lambda i: (i, 0)),
            pl.BlockSpec((1, 1), lambda i: (0, 0),
                         memory_space=pltpu.SMEM),
        ],
        out_shape=[
            jax.ShapeDtypeStruct((B, n), jnp.int32),
            jax.ShapeDtypeStruct((1, 1), jnp.float32),
        ],
    )(x, W1, b1.reshape(1, D), W2, b2.reshape(1, n * D), cb)
    return idx, losssum


_N_TOP_ROWS = B * 3
_N_GEO_ROWS = B * 4
_N_EXT_ROWS = B * 3
_NW = 32
_CH = 128


def _sc_gather_body(top_cb, geo_cb, ext_cb, it_h, ig_h, ie_h,
                    ot, og, oe, idx_v, rows_v, sem):
    wid = lax.axis_index("s") * 2 + lax.axis_index("c")
    for tab, idxh, oh, nrows in (
            (top_cb, it_h, ot, _N_TOP_ROWS),
            (geo_cb, ig_h, og, _N_GEO_ROWS),
            (ext_cb, ie_h, oe, _N_EXT_ROWS)):
        per = nrows // _NW
        base = wid * per

        def body(c, _):
            off = base + c * _CH
            pltpu.sync_copy(idxh.at[pl.ds(off, _CH)], idx_v)
            pltpu.async_copy(tab.at[idx_v], rows_v, sem).wait()
            pltpu.sync_copy(rows_v, oh.at[pl.ds(off, _CH)])
            return 0

        lax.fori_loop(0, per // _CH, body, 0)


def _sc_gather(top_cb, geo_cb, ext_cb, it, ig, ie):
    mesh = plsc.VectorSubcoreMesh(core_axis_name="c", subcore_axis_name="s")
    fn = functools.partial(
        pl.kernel, mesh=mesh,
        out_type=[
            jax.ShapeDtypeStruct((_N_TOP_ROWS, D), jnp.float32),
            jax.ShapeDtypeStruct((_N_GEO_ROWS, D), jnp.float32),
            jax.ShapeDtypeStruct((_N_EXT_ROWS, D), jnp.float32),
        ],
        scratch_types=[
            pltpu.VMEM((_CH,), jnp.int32),
            pltpu.VMEM((_CH, D), jnp.float32),
            pltpu.SemaphoreType.DMA,
        ],
    )(_sc_gather_body)
    return fn(top_cb, geo_cb, ext_cb, it, ig, ie)


def kernel(topology_features, geometry_features, extrusion_features,
           top_W1, top_b1, top_W2, top_b2,
           geo_W1, geo_b1, geo_W2, geo_b2,
           ext_W1, ext_b1, ext_W2, ext_b2,
           top_codebook, geo_codebook, ext_codebook):
    it2, st = _stream_tc(topology_features, top_W1, top_b1, top_W2, top_b2,
                         top_codebook, 3, 512)
    ig2, sg = _stream_tc(geometry_features, geo_W1, geo_b1, geo_W2, geo_b2,
                         geo_codebook, 4, 1024)
    ie2, se = _stream_tc(extrusion_features, ext_W1, ext_b1, ext_W2, ext_b2,
                         ext_codebook, 3, 1024)
    it = it2.reshape(_N_TOP_ROWS)
    ig = ig2.reshape(_N_GEO_ROWS)
    ie = ie2.reshape(_N_EXT_ROWS)
    qt, qg, qe = _sc_gather(top_codebook, geo_codebook, ext_codebook,
                            it, ig, ie)
    quantized = jnp.concatenate(
        [qt.reshape(B, 3, D), qg.reshape(B, 4, D), qe.reshape(B, 3, D)],
        axis=1)
    total_loss = 0.25 * (st[0, 0] / (_N_TOP_ROWS * D)
                         + sg[0, 0] / (_N_GEO_ROWS * D)
                         + se[0, 0] / (_N_EXT_ROWS * D))
    return quantized, total_loss, it, ig, ie


# speedup vs baseline: 2.5422x; 1.2173x over previous
import functools

import jax
import jax.numpy as jnp
from jax import lax
from jax.experimental import pallas as pl
from jax.experimental.pallas import tpu as pltpu
from jax.experimental.pallas import tpu_sc as plsc

B = 16384
D = 256
TB = 512


def _vq_body(n, K, x_ref, w1_ref, b1_ref, w2_ref, b2_ref, cb_ref,
             idx_ref, loss_ref):
    i = pl.program_id(0)
    x = x_ref[...]
    h = jnp.maximum(
        jnp.dot(x, w1_ref[...], preferred_element_type=jnp.float32)
        + b1_ref[...], 0.0)
    z = (jnp.dot(h, w2_ref[...], preferred_element_type=jnp.float32)
         + b2_ref[...])
    cb = cb_ref[...]
    cbn = jnp.sum(cb * cb, axis=1)
    acc = jnp.zeros((), jnp.float32)
    cols = []
    for j in range(n):
        zj = z[:, j * D:(j + 1) * D]
        zn = jnp.sum(zj * zj, axis=1, keepdims=True)
        mm = lax.dot_general(zj, cb, (((1,), (1,)), ((), ())),
                             preferred_element_type=jnp.float32)
        d = zn + cbn[None, :] - 2.0 * mm
        m = jnp.min(d, axis=1, keepdims=True)
        iota = lax.broadcasted_iota(jnp.int32, d.shape, 1)
        aj = jnp.min(jnp.where(d == m, iota, K), axis=1, keepdims=True)
        cols.append(aj)
        acc = acc + jnp.sum(m)
    idx_ref[...] = jnp.concatenate(cols, axis=1)

    @pl.when(i == 0)
    def _init():
        loss_ref[0, 0] = 0.0

    loss_ref[0, 0] += acc


def _stream_tc(x, W1, b1, W2, b2, cb, n, K):
    grid = (B // TB,)
    idx, losssum = pl.pallas_call(
        functools.partial(_vq_body, n, K),
        grid=grid,
        in_specs=[
            pl.BlockSpec((TB, D), lambda i: (i, 0)),
            pl.BlockSpec((D, D), lambda i: (0, 0)),
            pl.BlockSpec((1, D), lambda i: (0, 0)),
            pl.BlockSpec((D, n * D), lambda i: (0, 0)),
            pl.BlockSpec((1, n * D), lambda i: (0, 0)),
            pl.BlockSpec((K, D), lambda i: (0, 0)),
        ],
        out_specs=[
            pl.BlockSpec((TB, n), lambda i: (i, 0)),
            pl.BlockSpec((1, 1), lambda i: (0, 0),
                         memory_space=pltpu.SMEM),
        ],
        out_shape=[
            jax.ShapeDtypeStruct((B, n), jnp.int32),
            jax.ShapeDtypeStruct((1, 1), jnp.float32),
        ],
    )(x, W1, b1.reshape(1, D), W2, b2.reshape(1, n * D), cb)
    return idx, losssum


_N_TOP_ROWS = B * 3
_N_GEO_ROWS = B * 4
_N_EXT_ROWS = B * 3
_N_ALL_ROWS = B * 10
_NW = 32
_CH = 128
_PER_W = _N_ALL_ROWS // _NW
_NCH = _PER_W // _CH


def _sc_gather_body(tab, idx_h, out_h,
                    idx_a, idx_b, rows_a, rows_b, sem_a, sem_b):
    wid = lax.axis_index("s") * 2 + lax.axis_index("c")
    base = wid * _PER_W

    pltpu.sync_copy(idx_h.at[pl.ds(base, _CH)], idx_a)
    cp_a = pltpu.async_copy(tab.at[idx_a], rows_a, sem_a)

    def body(cc, _):
        o1 = base + (2 * cc + 1) * _CH
        pltpu.sync_copy(idx_h.at[pl.ds(o1, _CH)], idx_b)
        cp_b = pltpu.async_copy(tab.at[idx_b], rows_b, sem_b)
        o0 = base + (2 * cc) * _CH
        pltpu.make_async_copy(tab.at[idx_a], rows_a, sem_a).wait()
        pltpu.sync_copy(rows_a, out_h.at[pl.ds(o0, _CH)])

        @pl.when(cc + 1 < _NCH // 2)
        def _next():
            o2 = base + (2 * cc + 2) * _CH
            pltpu.sync_copy(idx_h.at[pl.ds(o2, _CH)], idx_a)
            pltpu.async_copy(tab.at[idx_a], rows_a, sem_a)

        cp_b.wait()
        pltpu.sync_copy(rows_b, out_h.at[pl.ds(o1, _CH)])
        return 0

    lax.fori_loop(0, _NCH // 2, body, 0)


def _sc_gather(tab, idx_all):
    mesh = plsc.VectorSubcoreMesh(core_axis_name="c", subcore_axis_name="s")
    fn = functools.partial(
        pl.kernel, mesh=mesh,
        out_type=jax.ShapeDtypeStruct((_N_ALL_ROWS, D), jnp.float32),
        scratch_types=[
            pltpu.VMEM((_CH,), jnp.int32),
            pltpu.VMEM((_CH,), jnp.int32),
            pltpu.VMEM((_CH, D), jnp.float32),
            pltpu.VMEM((_CH, D), jnp.float32),
            pltpu.SemaphoreType.DMA,
            pltpu.SemaphoreType.DMA,
        ],
    )(_sc_gather_body)
    return fn(tab, idx_all)


def kernel(topology_features, geometry_features, extrusion_features,
           top_W1, top_b1, top_W2, top_b2,
           geo_W1, geo_b1, geo_W2, geo_b2,
           ext_W1, ext_b1, ext_W2, ext_b2,
           top_codebook, geo_codebook, ext_codebook):
    it2, st = _stream_tc(topology_features, top_W1, top_b1, top_W2, top_b2,
                         top_codebook, 3, 512)
    ig2, sg = _stream_tc(geometry_features, geo_W1, geo_b1, geo_W2, geo_b2,
                         geo_codebook, 4, 1024)
    ie2, se = _stream_tc(extrusion_features, ext_W1, ext_b1, ext_W2, ext_b2,
                         ext_codebook, 3, 1024)
    it = it2.reshape(_N_TOP_ROWS)
    ig = ig2.reshape(_N_GEO_ROWS)
    ie = ie2.reshape(_N_EXT_ROWS)
    tab = jnp.concatenate([top_codebook, geo_codebook, ext_codebook], axis=0)
    idx_all = jnp.concatenate(
        [it2, ig2 + 512, ie2 + 1536], axis=1).reshape(_N_ALL_ROWS)
    q = _sc_gather(tab, idx_all)
    quantized = q.reshape(B, 10, D)
    total_loss = 0.25 * (st[0, 0] / (_N_TOP_ROWS * D)
                         + sg[0, 0] / (_N_GEO_ROWS * D)
                         + se[0, 0] / (_N_EXT_ROWS * D))
    return quantized, total_loss, it, ig, ie


# speedup vs baseline: 2.5433x; 1.0005x over previous
import functools

import jax
import jax.numpy as jnp
from jax import lax
from jax.experimental import pallas as pl
from jax.experimental.pallas import tpu as pltpu
from jax.experimental.pallas import tpu_sc as plsc

B = 16384
D = 256
TB = 512


def _vq_body(n, K, x_ref, w1_ref, b1_ref, w2_ref, b2_ref, cb_ref,
             idx_ref, loss_ref):
    i = pl.program_id(0)
    x = x_ref[...]
    h = jnp.maximum(
        jnp.dot(x, w1_ref[...], preferred_element_type=jnp.float32)
        + b1_ref[...], 0.0)
    z = (jnp.dot(h, w2_ref[...], preferred_element_type=jnp.float32)
         + b2_ref[...])
    cb = cb_ref[...]
    cbn = jnp.sum(cb * cb, axis=1)
    acc = jnp.zeros((), jnp.float32)
    cols = []
    for j in range(n):
        zj = z[:, j * D:(j + 1) * D]
        zn = jnp.sum(zj * zj, axis=1, keepdims=True)
        mm = lax.dot_general(zj, cb, (((1,), (1,)), ((), ())),
                             preferred_element_type=jnp.float32)
        d = zn + cbn[None, :] - 2.0 * mm
        m = jnp.min(d, axis=1, keepdims=True)
        iota = lax.broadcasted_iota(jnp.int32, d.shape, 1)
        aj = jnp.min(jnp.where(d == m, iota, K), axis=1, keepdims=True)
        cols.append(aj)
        acc = acc + jnp.sum(m)
    idx_ref[...] = jnp.concatenate(cols, axis=1)

    @pl.when(i == 0)
    def _init():
        loss_ref[0, 0] = 0.0

    loss_ref[0, 0] += acc


def _stream_tc(x, W1, b1, W2, b2, cb, n, K):
    grid = (B // TB,)
    idx, losssum = pl.pallas_call(
        functools.partial(_vq_body, n, K),
        grid=grid,
        in_specs=[
            pl.BlockSpec((TB, D), lambda i: (i, 0)),
            pl.BlockSpec((D, D), lambda i: (0, 0)),
            pl.BlockSpec((1, D), lambda i: (0, 0)),
            pl.BlockSpec((D, n * D), lambda i: (0, 0)),
            pl.BlockSpec((1, n * D), lambda i: (0, 0)),
            pl.BlockSpec((K, D), lambda i: (0, 0)),
        ],
        out_specs=[
            pl.BlockSpec((TB, n), lambda i: (i, 0)),
            pl.BlockSpec((1, 1), lambda i: (0, 0),
                         memory_space=pltpu.SMEM),
        ],
        out_shape=[
            jax.ShapeDtypeStruct((B, n), jnp.int32),
            jax.ShapeDtypeStruct((1, 1), jnp.float32),
        ],
    )(x, W1, b1.reshape(1, D), W2, b2.reshape(1, n * D), cb)
    return idx, losssum


_N_TOP_ROWS = B * 3
_N_GEO_ROWS = B * 4
_N_EXT_ROWS = B * 3
_N_ALL_ROWS = B * 10
_NW = 32
_CH = 128
_PER_W = _N_ALL_ROWS // _NW
_NCH = _PER_W // _CH


def _sc_gather_body(tab, idx_h, out_h,
                    idx_a, idx_b, rows_a, rows_b, sem_a, sem_b):
    wid = lax.axis_index("s") * 2 + lax.axis_index("c")
    base = wid * _PER_W

    pltpu.sync_copy(idx_h.at[pl.ds(base, _CH)], idx_a)
    cp_a = pltpu.async_copy(tab.at[idx_a], rows_a, sem_a)

    def body(cc, _):
        o1 = base + (2 * cc + 1) * _CH
        pltpu.sync_copy(idx_h.at[pl.ds(o1, _CH)], idx_b)
        cp_b = pltpu.async_copy(tab.at[idx_b], rows_b, sem_b)
        o0 = base + (2 * cc) * _CH
        pltpu.make_async_copy(tab.at[idx_a], rows_a, sem_a).wait()
        pltpu.sync_copy(rows_a, out_h.at[pl.ds(o0, _CH)])

        @pl.when(cc + 1 < _NCH // 2)
        def _next():
            o2 = base + (2 * cc + 2) * _CH
            pltpu.sync_copy(idx_h.at[pl.ds(o2, _CH)], idx_a)
            pltpu.async_copy(tab.at[idx_a], rows_a, sem_a)

        cp_b.wait()
        pltpu.sync_copy(rows_b, out_h.at[pl.ds(o1, _CH)])
        return 0

    lax.fori_loop(0, _NCH // 2, body, 0)


def _sc_gather(tab, idx_all):
    mesh = plsc.VectorSubcoreMesh(core_axis_name="c", subcore_axis_name="s")
    fn = functools.partial(
        pl.kernel, mesh=mesh,
        compiler_params=pltpu.CompilerParams(use_tc_tiling_on_sc=True),
        out_type=jax.ShapeDtypeStruct((_N_ALL_ROWS, D), jnp.float32),
        scratch_types=[
            pltpu.VMEM((_CH,), jnp.int32),
            pltpu.VMEM((_CH,), jnp.int32),
            pltpu.VMEM((_CH, D), jnp.float32),
            pltpu.VMEM((_CH, D), jnp.float32),
            pltpu.SemaphoreType.DMA,
            pltpu.SemaphoreType.DMA,
        ],
    )(_sc_gather_body)
    return fn(tab, idx_all)


def kernel(topology_features, geometry_features, extrusion_features,
           top_W1, top_b1, top_W2, top_b2,
           geo_W1, geo_b1, geo_W2, geo_b2,
           ext_W1, ext_b1, ext_W2, ext_b2,
           top_codebook, geo_codebook, ext_codebook):
    it2, st = _stream_tc(topology_features, top_W1, top_b1, top_W2, top_b2,
                         top_codebook, 3, 512)
    ig2, sg = _stream_tc(geometry_features, geo_W1, geo_b1, geo_W2, geo_b2,
                         geo_codebook, 4, 1024)
    ie2, se = _stream_tc(extrusion_features, ext_W1, ext_b1, ext_W2, ext_b2,
                         ext_codebook, 3, 1024)
    it = it2.reshape(_N_TOP_ROWS)
    ig = ig2.reshape(_N_GEO_ROWS)
    ie = ie2.reshape(_N_EXT_ROWS)
    tab = jnp.concatenate([top_codebook, geo_codebook, ext_codebook], axis=0)
    idx_all = jnp.concatenate(
        [it2, ig2 + 512, ie2 + 1536], axis=1).reshape(_N_ALL_ROWS)
    q = _sc_gather(tab, idx_all)
    quantized = q.reshape(B, 10, D)
    total_loss = 0.25 * (st[0, 0] / (_N_TOP_ROWS * D)
                         + sg[0, 0] / (_N_GEO_ROWS * D)
                         + se[0, 0] / (_N_EXT_ROWS * D))
    return quantized, total_loss, it, ig, ie


# speedup vs baseline: 2.5829x; 1.0155x over previous
import functools

import jax
import jax.numpy as jnp
from jax import lax
from jax.experimental import pallas as pl
from jax.experimental.pallas import tpu as pltpu
from jax.experimental.pallas import tpu_sc as plsc

B = 16384
D = 256
TB = 512


def _vq_body(n, K, x_ref, w1_ref, b1_ref, w2_ref, b2_ref, cb_ref,
             idx_ref, loss_ref):
    i = pl.program_id(0)
    x = x_ref[...]
    h = jnp.maximum(
        jnp.dot(x, w1_ref[...], preferred_element_type=jnp.float32)
        + b1_ref[...], 0.0)
    z = (jnp.dot(h, w2_ref[...], preferred_element_type=jnp.float32)
         + b2_ref[...])
    cb = cb_ref[...]
    cbn = jnp.sum(cb * cb, axis=1)
    acc = jnp.zeros((), jnp.float32)
    cols = []
    for j in range(n):
        zj = z[:, j * D:(j + 1) * D]
        zn = jnp.sum(zj * zj, axis=1, keepdims=True)
        mm = lax.dot_general(zj, cb, (((1,), (1,)), ((), ())),
                             preferred_element_type=jnp.float32)
        d = zn + cbn[None, :] - 2.0 * mm
        m = jnp.min(d, axis=1, keepdims=True)
        iota = lax.broadcasted_iota(jnp.int32, d.shape, 1)
        aj = jnp.min(jnp.where(d == m, iota, K), axis=1, keepdims=True)
        cols.append(aj)
        acc = acc + jnp.sum(m)
    idx_ref[...] = jnp.concatenate(cols, axis=1)

    @pl.when(i == 0)
    def _init():
        loss_ref[0, 0] = 0.0

    loss_ref[0, 0] += acc


def _stream_tc(x, W1, b1, W2, b2, cb, n, K):
    grid = (B // TB,)
    idx, losssum = pl.pallas_call(
        functools.partial(_vq_body, n, K),
        grid=grid,
        in_specs=[
            pl.BlockSpec((TB, D), lambda i: (i, 0)),
            pl.BlockSpec((D, D), lambda i: (0, 0)),
            pl.BlockSpec((1, D), lambda i: (0, 0)),
            pl.BlockSpec((D, n * D), lambda i: (0, 0)),
            pl.BlockSpec((1, n * D), lambda i: (0, 0)),
            pl.BlockSpec((K, D), lambda i: (0, 0)),
        ],
        out_specs=[
            pl.BlockSpec((TB, n), lambda i: (i, 0)),
            pl.BlockSpec((1, 1), lambda i: (0, 0),
                         memory_space=pltpu.SMEM),
        ],
        out_shape=[
            jax.ShapeDtypeStruct((B, n), jnp.int32),
            jax.ShapeDtypeStruct((1, 1), jnp.float32),
        ],
    )(x, W1, b1.reshape(1, D), W2, b2.reshape(1, n * D), cb)
    return idx, losssum


_N_TOP_ROWS = B * 3
_N_GEO_ROWS = B * 4
_N_EXT_ROWS = B * 3
_N_ALL_ROWS = B * 10
_NW = 32
_CH = 128
_PER_W = _N_ALL_ROWS // _NW
_NCH = _PER_W // _CH


def _sc_gather_body(tab, idx_h, out_h,
                    idx_a, idx_b, rows_a, rows_b, sem_a, sem_b):
    wid = lax.axis_index("s") * 2 + lax.axis_index("c")
    base = wid * _PER_W

    pltpu.sync_copy(idx_h.at[pl.ds(base, _CH)], idx_a)
    pltpu.async_copy(tab.at[idx_a], rows_a, sem_a)

    def body(cc, _):
        o1 = base + (2 * cc + 1) * _CH
        pltpu.sync_copy(idx_h.at[pl.ds(o1, _CH)], idx_b)
        cp_b = pltpu.async_copy(tab.at[idx_b], rows_b, sem_b)
        o0 = base + (2 * cc) * _CH
        pltpu.make_async_copy(tab.at[idx_a], rows_a, sem_a).wait()
        pltpu.sync_copy(rows_a, out_h.at[pl.ds(o0, _CH)])

        @pl.when(cc + 1 < _NCH // 2)
        def _next():
            o2 = base + (2 * cc + 2) * _CH
            pltpu.sync_copy(idx_h.at[pl.ds(o2, _CH)], idx_a)
            pltpu.async_copy(tab.at[idx_a], rows_a, sem_a)

        cp_b.wait()
        pltpu.sync_copy(rows_b, out_h.at[pl.ds(o1, _CH)])
        return 0

    lax.fori_loop(0, _NCH // 2, body, 0)


def _sc_gather(tab, idx_all):
    mesh = plsc.VectorSubcoreMesh(core_axis_name="c", subcore_axis_name="s")
    fn = functools.partial(
        pl.kernel, mesh=mesh,
        out_type=jax.ShapeDtypeStruct((_N_ALL_ROWS, D), jnp.float32),
        scratch_types=[
            pltpu.VMEM((_CH,), jnp.int32),
            pltpu.VMEM((_CH,), jnp.int32),
            pltpu.VMEM((_CH, D), jnp.float32),
            pltpu.VMEM((_CH, D), jnp.float32),
            pltpu.SemaphoreType.DMA,
            pltpu.SemaphoreType.DMA,
        ],
    )(_sc_gather_body)
    return fn(tab, idx_all)


_TQ = 256


def _repack_body(in_ref, out_ref):
    out_ref[...] = in_ref[...].reshape(_TQ, 10, D)


def _repack(q2d):
    return pl.pallas_call(
        _repack_body,
        grid=(B // _TQ,),
        in_specs=[pl.BlockSpec((_TQ * 10, D), lambda i: (i, 0))],
        out_specs=pl.BlockSpec((_TQ, 10, D), lambda i: (i, 0, 0)),
        out_shape=jax.ShapeDtypeStruct((B, 10, D), jnp.float32),
    )(q2d)


def kernel(topology_features, geometry_features, extrusion_features,
           top_W1, top_b1, top_W2, top_b2,
           geo_W1, geo_b1, geo_W2, geo_b2,
           ext_W1, ext_b1, ext_W2, ext_b2,
           top_codebook, geo_codebook, ext_codebook):
    it2, st = _stream_tc(topology_features, top_W1, top_b1, top_W2, top_b2,
                         top_codebook, 3, 512)
    ig2, sg = _stream_tc(geometry_features, geo_W1, geo_b1, geo_W2, geo_b2,
                         geo_codebook, 4, 1024)
    ie2, se = _stream_tc(extrusion_features, ext_W1, ext_b1, ext_W2, ext_b2,
                         ext_codebook, 3, 1024)
    it = it2.reshape(_N_TOP_ROWS)
    ig = ig2.reshape(_N_GEO_ROWS)
    ie = ie2.reshape(_N_EXT_ROWS)
    tab = jnp.concatenate([top_codebook, geo_codebook, ext_codebook], axis=0)
    idx_all = jnp.concatenate(
        [it2, ig2 + 512, ie2 + 1536], axis=1).reshape(_N_ALL_ROWS)
    q = _sc_gather(tab, idx_all)
    quantized = _repack(q)
    total_loss = 0.25 * (st[0, 0] / (_N_TOP_ROWS * D)
                         + sg[0, 0] / (_N_GEO_ROWS * D)
                         + se[0, 0] / (_N_EXT_ROWS * D))
    return quantized, total_loss, it, ig, ie


# speedup vs baseline: 3.0088x; 1.1649x over previous
import functools

import jax
import jax.numpy as jnp
from jax import lax
from jax.experimental import pallas as pl
from jax.experimental.pallas import tpu as pltpu
from jax.experimental.pallas import tpu_sc as plsc

B = 16384
H = 2
BH = B // H
D = 256
TB = 512


def _vq_body(n, K, x_ref, w1_ref, b1_ref, w2_ref, b2_ref, cb_ref,
             idx_ref, loss_ref):
    i = pl.program_id(0)
    x = x_ref[...]
    h = jnp.maximum(
        jnp.dot(x, w1_ref[...], preferred_element_type=jnp.float32)
        + b1_ref[...], 0.0)
    z = (jnp.dot(h, w2_ref[...], preferred_element_type=jnp.float32)
         + b2_ref[...])
    cb = cb_ref[...]
    cbn = jnp.sum(cb * cb, axis=1)
    acc = jnp.zeros((), jnp.float32)
    cols = []
    for j in range(n):
        zj = z[:, j * D:(j + 1) * D]
        zn = jnp.sum(zj * zj, axis=1, keepdims=True)
        mm = lax.dot_general(zj, cb, (((1,), (1,)), ((), ())),
                             preferred_element_type=jnp.float32)
        d = zn + cbn[None, :] - 2.0 * mm
        m = jnp.min(d, axis=1, keepdims=True)
        iota = lax.broadcasted_iota(jnp.int32, d.shape, 1)
        aj = jnp.min(jnp.where(d == m, iota, K), axis=1, keepdims=True)
        cols.append(aj)
        acc = acc + jnp.sum(m)
    idx_ref[...] = jnp.concatenate(cols, axis=1)

    @pl.when(i == 0)
    def _init():
        loss_ref[0, 0] = 0.0

    loss_ref[0, 0] += acc


def _stream_tc(x, W1, b1, W2, b2, cb, n, K, h):
    base = h * (BH // TB)
    idx, losssum = pl.pallas_call(
        functools.partial(_vq_body, n, K),
        grid=(BH // TB,),
        in_specs=[
            pl.BlockSpec((TB, D), lambda i: (base + i, 0)),
            pl.BlockSpec((D, D), lambda i: (0, 0)),
            pl.BlockSpec((1, D), lambda i: (0, 0)),
            pl.BlockSpec((D, n * D), lambda i: (0, 0)),
            pl.BlockSpec((1, n * D), lambda i: (0, 0)),
            pl.BlockSpec((K, D), lambda i: (0, 0)),
        ],
        out_specs=[
            pl.BlockSpec((TB, n), lambda i: (i, 0)),
            pl.BlockSpec((1, 1), lambda i: (0, 0),
                         memory_space=pltpu.SMEM),
        ],
        out_shape=[
            jax.ShapeDtypeStruct((BH, n), jnp.int32),
            jax.ShapeDtypeStruct((1, 1), jnp.float32),
        ],
    )(x, W1, b1.reshape(1, D), W2, b2.reshape(1, n * D), cb)
    return idx, losssum


_N_TOP_ROWS = B * 3
_N_GEO_ROWS = B * 4
_N_EXT_ROWS = B * 3
_NH_ROWS = BH * 10
_NW = 32
_CH = 128
_PER_W = _NH_ROWS // _NW
_NCH = _PER_W // _CH


def _sc_gather_body(tab, idx_h, out_h,
                    idx_a, idx_b, rows_a, rows_b, sem_a, sem_b):
    wid = lax.axis_index("s") * 2 + lax.axis_index("c")
    base = wid * _PER_W

    pltpu.sync_copy(idx_h.at[pl.ds(base, _CH)], idx_a)
    pltpu.async_copy(tab.at[idx_a], rows_a, sem_a)

    def body(cc, _):
        o1 = base + (2 * cc + 1) * _CH
        pltpu.sync_copy(idx_h.at[pl.ds(o1, _CH)], idx_b)
        cp_b = pltpu.async_copy(tab.at[idx_b], rows_b, sem_b)
        o0 = base + (2 * cc) * _CH
        pltpu.make_async_copy(tab.at[idx_a], rows_a, sem_a).wait()
        pltpu.sync_copy(rows_a, out_h.at[pl.ds(o0, _CH)])

        @pl.when(cc + 1 < _NCH // 2)
        def _next():
            o2 = base + (2 * cc + 2) * _CH
            pltpu.sync_copy(idx_h.at[pl.ds(o2, _CH)], idx_a)
            pltpu.async_copy(tab.at[idx_a], rows_a, sem_a)

        cp_b.wait()
        pltpu.sync_copy(rows_b, out_h.at[pl.ds(o1, _CH)])
        return 0

    lax.fori_loop(0, _NCH // 2, body, 0)


def _sc_gather(tab, idx_half):
    mesh = plsc.VectorSubcoreMesh(core_axis_name="c", subcore_axis_name="s")
    fn = functools.partial(
        pl.kernel, mesh=mesh,
        out_type=jax.ShapeDtypeStruct((_NH_ROWS, D), jnp.float32),
        scratch_types=[
            pltpu.VMEM((_CH,), jnp.int32),
            pltpu.VMEM((_CH,), jnp.int32),
            pltpu.VMEM((_CH, D), jnp.float32),
            pltpu.VMEM((_CH, D), jnp.float32),
            pltpu.SemaphoreType.DMA,
            pltpu.SemaphoreType.DMA,
        ],
    )(_sc_gather_body)
    return fn(tab, idx_half)


_TQ = 256


def _repack_first_body(in_ref, out_ref):
    out_ref[...] = in_ref[...].reshape(_TQ, 10, D)


def _repack_rest_body(in_ref, alias_ref, out_ref):
    out_ref[...] = in_ref[...].reshape(_TQ, 10, D)


def _repack_first(q_half):
    return pl.pallas_call(
        _repack_first_body,
        grid=(BH // _TQ,),
        in_specs=[pl.BlockSpec((_TQ * 10, D), lambda i: (i, 0))],
        out_specs=pl.BlockSpec((_TQ, 10, D), lambda i: (i, 0, 0)),
        out_shape=jax.ShapeDtypeStruct((B, 10, D), jnp.float32),
    )(q_half)


def _repack_rest(q_half, partial_out, h):
    base = h * (BH // _TQ)
    return pl.pallas_call(
        _repack_rest_body,
        grid=(BH // _TQ,),
        in_specs=[
            pl.BlockSpec((_TQ * 10, D), lambda i: (i, 0)),
            pl.BlockSpec(memory_space=pl.ANY),
        ],
        out_specs=pl.BlockSpec((_TQ, 10, D), lambda i: (base + i, 0, 0)),
        out_shape=jax.ShapeDtypeStruct((B, 10, D), jnp.float32),
        input_output_aliases={1: 0},
    )(q_half, partial_out)


def kernel(topology_features, geometry_features, extrusion_features,
           top_W1, top_b1, top_W2, top_b2,
           geo_W1, geo_b1, geo_W2, geo_b2,
           ext_W1, ext_b1, ext_W2, ext_b2,
           top_codebook, geo_codebook, ext_codebook):
    tab = jnp.concatenate([top_codebook, geo_codebook, ext_codebook], axis=0)
    its, igs, ies, losses, qs = [], [], [], [], []
    for h in range(H):
        it2, st = _stream_tc(topology_features, top_W1, top_b1, top_W2,
                             top_b2, top_codebook, 3, 512, h)
        ig2, sg = _stream_tc(geometry_features, geo_W1, geo_b1, geo_W2,
                             geo_b2, geo_codebook, 4, 1024, h)
        ie2, se = _stream_tc(extrusion_features, ext_W1, ext_b1, ext_W2,
                             ext_b2, ext_codebook, 3, 1024, h)
        idx_half = jnp.concatenate(
            [it2, ig2 + 512, ie2 + 1536], axis=1).reshape(_NH_ROWS)
        qs.append(_sc_gather(tab, idx_half))
        its.append(it2)
        igs.append(ig2)
        ies.append(ie2)
        losses.append(0.25 * (st[0, 0] / (_N_TOP_ROWS * D)
                              + sg[0, 0] / (_N_GEO_ROWS * D)
                              + se[0, 0] / (_N_EXT_ROWS * D)))
    quantized = _repack_first(qs[0])
    for h in range(1, H):
        quantized = _repack_rest(qs[h], quantized, h)
    it = jnp.concatenate(its, axis=0).reshape(_N_TOP_ROWS)
    ig = jnp.concatenate(igs, axis=0).reshape(_N_GEO_ROWS)
    ie = jnp.concatenate(ies, axis=0).reshape(_N_EXT_ROWS)
    total_loss = sum(losses)
    return quantized, total_loss, it, ig, ie


# speedup vs baseline: 3.0660x; 1.0190x over previous
import functools

import jax
import jax.numpy as jnp
from jax import lax
from jax.experimental import pallas as pl
from jax.experimental.pallas import tpu as pltpu
from jax.experimental.pallas import tpu_sc as plsc

B = 16384
H = 4
BH = B // H
D = 256
TB = 512


def _vq_body(n, K, x_ref, w1_ref, b1_ref, w2_ref, b2_ref, cb_ref,
             idx_ref, loss_ref):
    i = pl.program_id(0)
    x = x_ref[...]
    h = jnp.maximum(
        jnp.dot(x, w1_ref[...], preferred_element_type=jnp.float32)
        + b1_ref[...], 0.0)
    z = (jnp.dot(h, w2_ref[...], preferred_element_type=jnp.float32)
         + b2_ref[...])
    cb = cb_ref[...]
    cbn = jnp.sum(cb * cb, axis=1)
    acc = jnp.zeros((), jnp.float32)
    cols = []
    for j in range(n):
        zj = z[:, j * D:(j + 1) * D]
        zn = jnp.sum(zj * zj, axis=1, keepdims=True)
        mm = lax.dot_general(zj, cb, (((1,), (1,)), ((), ())),
                             preferred_element_type=jnp.float32)
        d = zn + cbn[None, :] - 2.0 * mm
        m = jnp.min(d, axis=1, keepdims=True)
        iota = lax.broadcasted_iota(jnp.int32, d.shape, 1)
        aj = jnp.min(jnp.where(d == m, iota, K), axis=1, keepdims=True)
        cols.append(aj)
        acc = acc + jnp.sum(m)
    idx_ref[...] = jnp.concatenate(cols, axis=1)

    @pl.when(i == 0)
    def _init():
        loss_ref[0, 0] = 0.0

    loss_ref[0, 0] += acc


def _stream_tc(x, W1, b1, W2, b2, cb, n, K, h):
    base = h * (BH // TB)
    idx, losssum = pl.pallas_call(
        functools.partial(_vq_body, n, K),
        grid=(BH // TB,),
        in_specs=[
            pl.BlockSpec((TB, D), lambda i: (base + i, 0)),
            pl.BlockSpec((D, D), lambda i: (0, 0)),
            pl.BlockSpec((1, D), lambda i: (0, 0)),
            pl.BlockSpec((D, n * D), lambda i: (0, 0)),
            pl.BlockSpec((1, n * D), lambda i: (0, 0)),
            pl.BlockSpec((K, D), lambda i: (0, 0)),
        ],
        out_specs=[
            pl.BlockSpec((TB, n), lambda i: (i, 0)),
            pl.BlockSpec((1, 1), lambda i: (0, 0),
                         memory_space=pltpu.SMEM),
        ],
        out_shape=[
            jax.ShapeDtypeStruct((BH, n), jnp.int32),
            jax.ShapeDtypeStruct((1, 1), jnp.float32),
        ],
    )(x, W1, b1.reshape(1, D), W2, b2.reshape(1, n * D), cb)
    return idx, losssum


_N_TOP_ROWS = B * 3
_N_GEO_ROWS = B * 4
_N_EXT_ROWS = B * 3
_NH_ROWS = BH * 10
_NW = 32
_CH = 128
_PER_W = _NH_ROWS // _NW
_NCH = _PER_W // _CH


def _sc_gather_body(tab, idx_h, out_h,
                    idx_a, idx_b, rows_a, rows_b, sem_a, sem_b):
    wid = lax.axis_index("s") * 2 + lax.axis_index("c")
    base = wid * _PER_W

    pltpu.sync_copy(idx_h.at[pl.ds(base, _CH)], idx_a)
    pltpu.async_copy(tab.at[idx_a], rows_a, sem_a)

    def body(cc, _):
        o1 = base + (2 * cc + 1) * _CH
        pltpu.sync_copy(idx_h.at[pl.ds(o1, _CH)], idx_b)
        cp_b = pltpu.async_copy(tab.at[idx_b], rows_b, sem_b)
        o0 = base + (2 * cc) * _CH
        pltpu.make_async_copy(tab.at[idx_a], rows_a, sem_a).wait()
        pltpu.sync_copy(rows_a, out_h.at[pl.ds(o0, _CH)])

        @pl.when(cc + 1 < _NCH // 2)
        def _next():
            o2 = base + (2 * cc + 2) * _CH
            pltpu.sync_copy(idx_h.at[pl.ds(o2, _CH)], idx_a)
            pltpu.async_copy(tab.at[idx_a], rows_a, sem_a)

        cp_b.wait()
        pltpu.sync_copy(rows_b, out_h.at[pl.ds(o1, _CH)])
        return 0

    lax.fori_loop(0, _NCH // 2, body, 0)


def _sc_gather(tab, idx_half):
    mesh = plsc.VectorSubcoreMesh(core_axis_name="c", subcore_axis_name="s")
    fn = functools.partial(
        pl.kernel, mesh=mesh,
        out_type=jax.ShapeDtypeStruct((_NH_ROWS, D), jnp.float32),
        scratch_types=[
            pltpu.VMEM((_CH,), jnp.int32),
            pltpu.VMEM((_CH,), jnp.int32),
            pltpu.VMEM((_CH, D), jnp.float32),
            pltpu.VMEM((_CH, D), jnp.float32),
            pltpu.SemaphoreType.DMA,
            pltpu.SemaphoreType.DMA,
        ],
    )(_sc_gather_body)
    return fn(tab, idx_half)


_TQ = 256


def _repack_first_body(in_ref, out_ref):
    out_ref[...] = in_ref[...].reshape(_TQ, 10, D)


def _repack_rest_body(in_ref, alias_ref, out_ref):
    out_ref[...] = in_ref[...].reshape(_TQ, 10, D)


def _repack_first(q_half):
    return pl.pallas_call(
        _repack_first_body,
        grid=(BH // _TQ,),
        in_specs=[pl.BlockSpec((_TQ * 10, D), lambda i: (i, 0))],
        out_specs=pl.BlockSpec((_TQ, 10, D), lambda i: (i, 0, 0)),
        out_shape=jax.ShapeDtypeStruct((B, 10, D), jnp.float32),
    )(q_half)


def _repack_rest(q_half, partial_out, h):
    base = h * (BH // _TQ)
    return pl.pallas_call(
        _repack_rest_body,
        grid=(BH // _TQ,),
        in_specs=[
            pl.BlockSpec((_TQ * 10, D), lambda i: (i, 0)),
            pl.BlockSpec(memory_space=pl.ANY),
        ],
        out_specs=pl.BlockSpec((_TQ, 10, D), lambda i: (base + i, 0, 0)),
        out_shape=jax.ShapeDtypeStruct((B, 10, D), jnp.float32),
        input_output_aliases={1: 0},
    )(q_half, partial_out)


def kernel(topology_features, geometry_features, extrusion_features,
           top_W1, top_b1, top_W2, top_b2,
           geo_W1, geo_b1, geo_W2, geo_b2,
           ext_W1, ext_b1, ext_W2, ext_b2,
           top_codebook, geo_codebook, ext_codebook):
    tab = jnp.concatenate([top_codebook, geo_codebook, ext_codebook], axis=0)
    its, igs, ies, losses, qs = [], [], [], [], []
    for h in range(H):
        it2, st = _stream_tc(topology_features, top_W1, top_b1, top_W2,
                             top_b2, top_codebook, 3, 512, h)
        ig2, sg = _stream_tc(geometry_features, geo_W1, geo_b1, geo_W2,
                             geo_b2, geo_codebook, 4, 1024, h)
        ie2, se = _stream_tc(extrusion_features, ext_W1, ext_b1, ext_W2,
                             ext_b2, ext_codebook, 3, 1024, h)
        idx_half = jnp.concatenate(
            [it2, ig2 + 512, ie2 + 1536], axis=1).reshape(_NH_ROWS)
        qs.append(_sc_gather(tab, idx_half))
        its.append(it2)
        igs.append(ig2)
        ies.append(ie2)
        losses.append(0.25 * (st[0, 0] / (_N_TOP_ROWS * D)
                              + sg[0, 0] / (_N_GEO_ROWS * D)
                              + se[0, 0] / (_N_EXT_ROWS * D)))
    quantized = _repack_first(qs[0])
    for h in range(1, H):
        quantized = _repack_rest(qs[h], quantized, h)
    it = jnp.concatenate(its, axis=0).reshape(_N_TOP_ROWS)
    ig = jnp.concatenate(igs, axis=0).reshape(_N_GEO_ROWS)
    ie = jnp.concatenate(ies, axis=0).reshape(_N_EXT_ROWS)
    total_loss = sum(losses)
    return quantized, total_loss, it, ig, ie
